# baseline (speedup 1.0000x reference)
import functools

import numpy as np
import jax
import jax.numpy as jnp
from jax import lax
from jax.experimental import pallas as pl
from jax.experimental.pallas import tpu as pltpu
from jax.experimental.pallas import tpu_sc as plsc

N = 10000
E = 320000
H = 64

NT = 32
RPB = 320
NPAD = NT * RPB
DUMMY = RPB
CHUNK = 8000
FLUSH = 8192
CAP = E + FLUSH + 512
GB = 64
PKSHIFT = 14
PKMASK = (1 << PKSHIFT) - 1
DUMMY_PK = DUMMY << PKSHIFT

_MESH = dict(core_axis_name="c", subcore_axis_name="s")


def _make_lut():
    lut = np.zeros((256, 16), np.int32)
    for m in range(256):
        bits = [i for i in range(8) if (m >> i) & 1]
        lut[m, : len(bits)] = bits
        lut[m, 8] = len(bits)
    return jnp.asarray(lut).reshape(256 * 16)


def _widx():
    return lax.axis_index("s") * 2 + lax.axis_index("c")


def _tree_sum(v):
    pos = lax.iota(jnp.int32, 16)
    for k in (1, 2, 4, 8):
        v = v + jnp.take(v, pos ^ k)
    return v


def _bucket_edges(src, dst, lut):
    mesh = plsc.VectorSubcoreMesh(**_MESH)

    @functools.partial(
        pl.kernel,
        out_type=[
            jax.ShapeDtypeStruct((NT * CAP,), jnp.int32),
            jax.ShapeDtypeStruct((NT * 16,), jnp.int32),
        ],
        mesh=mesh,
        scratch_types=[
            [pltpu.VMEM((CHUNK,), jnp.int32)] * 2,
            [pltpu.VMEM((CHUNK,), jnp.int32)] * 2,
            pltpu.VMEM((FLUSH,), jnp.int32),
            pltpu.VMEM((256 * 16,), jnp.int32),
            pltpu.VMEM((16,), jnp.int32),
            [pltpu.SemaphoreType.DMA] * 2,
            [pltpu.SemaphoreType.DMA] * 2,
        ],
    )
    def k(src_h, dst_h, lut_h, bpack_h, counts_h, sv, dv, cp, lutv, cbuf,
          ssem, dsem):
        wid = _widx()
        lo = wid * RPB
        pos = lax.iota(jnp.int32, 16)
        pow2_8 = jnp.int32(1) << (pos & 7)
        NCH = E // CHUNK

        def s_copy(ch, p):
            return pltpu.make_async_copy(
                src_h.at[pl.ds(ch * CHUNK, CHUNK)], sv[p], ssem[p])

        def d_copy(ch, p):
            return pltpu.make_async_copy(
                dst_h.at[pl.ds(ch * CHUNK, CHUNK)], dv[p], dsem[p])

        s_copy(0, 0).start()
        d_copy(0, 0).start()
        pltpu.sync_copy(lut_h, lutv)

        def chunk_body(ch, off, p):
            @pl.when(ch + 1 < NCH)
            def _():
                s_copy(ch + 1, 1 - p).start()
                d_copy(ch + 1, 1 - p).start()

            s_copy(ch, p).wait()
            d_copy(ch, p).wait()

            def vec_body(t, cnt):
                for u in range(4):
                    sl = pl.ds(t * 64 + u * 16, 16)
                    svv = sv[p][sl]
                    dvv = dv[p][sl]
                    m = (dvv >= lo) & (dvv < lo + RPB)
                    pk = jnp.where(m, ((dvv - lo) << PKSHIFT) | svv, DUMMY_PK)
                    s = jnp.where(m, pow2_8, 0)
                    for kk in (1, 2, 4):
                        s = s + jnp.take(s, pos ^ kk)
                    mlo = s[0]
                    mhi = s[8]

                    def compact(cnt_in):
                        rlo = lutv[pl.ds(mlo * 16, 16)]
                        rhi = lutv[pl.ds(mhi * 16, 16)]
                        cnt_lo = rlo[8]
                        cnt_hi = rhi[8]
                        idx_hi = jnp.clip(pos - cnt_lo, 0, 15)
                        perm = jnp.where(pos < cnt_lo, rlo,
                                         jnp.take(rhi + 8, idx_hi))
                        perm = jnp.clip(perm, 0, 15)
                        cp[pl.ds(cnt_in, 16)] = jnp.take(pk, perm)
                        return cnt_in + cnt_lo + cnt_hi

                    cnt = lax.cond(mlo + mhi > 0, compact, lambda c: c, cnt)
                return cnt

            cnt = lax.fori_loop(0, CHUNK // 64, vec_body, 0)
            cp[pl.ds(cnt, 16)] = jnp.zeros((16,), jnp.int32) + DUMMY_PK
            cnt = (cnt + 7) & (-8)
            pltpu.sync_copy(
                cp, bpack_h.at[pl.ds(pl.multiple_of(wid * CAP + off, 8), FLUSH)])
            return off + cnt

        def chunk_pair(g, off):
            off = chunk_body(g * 2, off, 0)
            off = chunk_body(g * 2 + 1, off, 1)
            return off

        off = lax.fori_loop(0, (E // CHUNK) // 2, chunk_pair, 0)
        for q in range(4):
            cp[pl.ds(q * 16, 16)] = jnp.zeros((16,), jnp.int32) + DUMMY_PK
        pltpu.sync_copy(
            cp, bpack_h.at[pl.ds(pl.multiple_of(wid * CAP + off, 8), FLUSH)])
        cbuf[...] = jnp.zeros((16,), jnp.int32) + off
        pltpu.sync_copy(cbuf, counts_h.at[pl.ds(pl.multiple_of(wid * 16, 16), 16)])

    return k(src, dst, lut)


def _segment_max(h2, bpack, counts):
    F = h2.shape[1]
    mesh = plsc.VectorSubcoreMesh(**_MESH)

    @functools.partial(
        pl.kernel,
        out_type=jax.ShapeDtypeStruct((NPAD, F), jnp.float32),
        mesh=mesh,
        scratch_types=[
            [pltpu.VMEM((GB,), jnp.int32)] * 2,
            [pltpu.VMEM((GB,), jnp.int32)] * 2,
            [pltpu.VMEM((GB,), jnp.int32)] * 2,
            [pltpu.VMEM((GB, F), jnp.float32)] * 2,
            pltpu.VMEM((RPB + 1, F), jnp.float32),
            pltpu.VMEM((16,), jnp.int32),
            [pltpu.SemaphoreType.DMA] * 2,
            [pltpu.SemaphoreType.DMA] * 2,
        ],
    )
    def k(h2_h, bp_h, cnt_h, out_h, pkbuf, idx, lbuf, stage, acc, cbuf,
          pksem, gsem):
        wid = _widx()
        neg = jnp.full((16,), -jnp.inf, jnp.float32)

        pltpu.sync_copy(cnt_h.at[pl.ds(pl.multiple_of(wid * 16, 16), 16)], cbuf)
        m_total = cbuf[pl.ds(0, 16)][0]
        nb = (m_total + GB - 1) // GB

        def pk_copy(i, b):
            return pltpu.make_async_copy(
                bp_h.at[pl.ds(pl.multiple_of(wid * CAP + i * GB, 8), GB)],
                pkbuf[b], pksem[b])

        def gather(b):
            return pltpu.make_async_copy(h2_h.at[idx[b]], stage[b], gsem[b])

        def unpack_and_fire(b):
            for q in range(GB // 16):
                sl = pl.ds(q * 16, 16)
                pv = pkbuf[b][sl]
                idx[b][sl] = pv & PKMASK
                lbuf[b][sl] = pv >> PKSHIFT
            gather(b).start()

        @pl.when(nb > 0)
        def _():
            pk_copy(0, 0).start()

        @pl.when(nb > 1)
        def _():
            pk_copy(1, 1).start()

        @pl.when(nb > 0)
        def _():
            pk_copy(0, 0).wait()
            unpack_and_fire(0)

        def init_row(r, _):
            for kk in range(F // 16):
                acc[r, pl.ds(kk * 16, 16)] = neg
            return 0

        lax.fori_loop(0, RPB + 1, init_row, 0)

        def pair(g, _):
            for b in range(2):
                i = g * 2 + b

                @pl.when(i + 1 < nb)
                def _():
                    pk_copy(i + 1, 1 - b).wait()
                    unpack_and_fire(1 - b)

                @pl.when(i + 2 < nb)
                def _():
                    pk_copy(i + 2, b).start()

                @pl.when(i < nb)
                def _():
                    gather(b).wait()

                    def grp(q, _):
                        ldv = lbuf[b][pl.ds(q * 16, 16)]
                        for u in range(16):
                            ld = ldv[u]
                            jr = q * 16 + u
                            for kk in range(F // 16):
                                sl = pl.ds(kk * 16, 16)
                                acc[ld, sl] = jnp.maximum(acc[ld, sl],
                                                          stage[b][jr, sl])
                        return 0

                    lax.fori_loop(0, GB // 16, grp, 0)
            return 0

        lax.fori_loop(0, (nb + 1) // 2, pair, 0)

        def fix_row(r, _):
            for kk in range(F // 16):
                sl = pl.ds(kk * 16, 16)
                v = acc[r, sl]
                acc[r, sl] = jnp.where(v == -jnp.inf, 0.0, v)
            return 0

        lax.fori_loop(0, RPB, fix_row, 0)
        pltpu.sync_copy(acc.at[pl.ds(0, RPB)],
                        out_h.at[pl.ds(pl.multiple_of(wid * RPB, 8), RPB)])

    return k(h2, bpack, counts)


def _mlp_body(x_ref, w1_ref, b1_ref, g_ref, bt_ref, w2_ref, b2_ref, o_ref):
    h = jnp.dot(x_ref[...], w1_ref[...], preferred_element_type=jnp.float32)
    h = h + b1_ref[...]
    mu = jnp.mean(h, axis=1, keepdims=True)
    var = jnp.mean(jnp.square(h - mu), axis=1, keepdims=True)
    h = (h - mu) * lax.rsqrt(var + 1e-5) * g_ref[...] + bt_ref[...]
    h = jnp.maximum(h, 0.0)
    o_ref[...] = jnp.dot(h, w2_ref[...], preferred_element_type=jnp.float32) + b2_ref[...]


def _mlp(x, W1, b1, g, bt, W2, b2):
    n, c = x.shape
    BN = 400
    return pl.pallas_call(
        _mlp_body,
        grid=(n // BN,),
        in_specs=[
            pl.BlockSpec((BN, c), lambda i: (i, 0)),
            pl.BlockSpec((c, H), lambda i: (0, 0)),
            pl.BlockSpec((1, H), lambda i: (0, 0)),
            pl.BlockSpec((1, H), lambda i: (0, 0)),
            pl.BlockSpec((1, H), lambda i: (0, 0)),
            pl.BlockSpec((H, c), lambda i: (0, 0)),
            pl.BlockSpec((1, c), lambda i: (0, 0)),
        ],
        out_specs=pl.BlockSpec((BN, c), lambda i: (i, 0)),
        out_shape=jax.ShapeDtypeStruct((n, c), jnp.float32),
    )(x, W1, b1.reshape(1, H), g.reshape(1, H), bt.reshape(1, H), W2,
      b2.reshape(1, c))


def kernel(x, edge_index,
           W1_0, b1_0, g_0, bt_0, W2_0, b2_0,
           W1_1, b1_1, g_1, bt_1, W2_1, b2_1,
           W1_2, b1_2, g_2, bt_2, W2_2, b2_2):
    src = edge_index[0]
    dst = edge_index[1]
    bpack, counts = _bucket_edges(src, dst, _make_lut())
    params = [
        (W1_0, b1_0, g_0, bt_0, W2_0, b2_0),
        (W1_1, b1_1, g_1, bt_1, W2_1, b2_1),
        (W1_2, b1_2, g_2, bt_2, W2_2, b2_2),
    ]
    for W1, b1, g, bt, W2, b2 in params:
        h2 = _mlp(x, W1, b1, g, bt, W2, b2)
        c = h2.shape[1]
        if c <= 256:
            agg = _segment_max(h2, bpack, counts)[:N]
        else:
            parts = [
                _segment_max(h2[:, f0:f0 + 256], bpack, counts)[:N]
                for f0 in range(0, c, 256)
            ]
            agg = jnp.concatenate(parts, axis=1)
        x = jnp.concatenate([h2, agg], axis=1)
    return x


# speedup vs baseline: 1.0155x; 1.0155x over previous
import functools

import numpy as np
import jax
import jax.numpy as jnp
from jax import lax
from jax.experimental import pallas as pl
from jax.experimental.pallas import tpu as pltpu
from jax.experimental.pallas import tpu_sc as plsc

N = 10000
E = 320000
H = 64

NT = 32
RPB = 320
NPAD = NT * RPB
DUMMY = RPB
CHUNK = 16000
FLUSH = 16384
CAP = E + FLUSH + 512
GB = 64
PKSHIFT = 14
PKMASK = (1 << PKSHIFT) - 1
DUMMY_PK = DUMMY << PKSHIFT

_MESH = dict(core_axis_name="c", subcore_axis_name="s")


def _make_lut():
    lut = np.zeros((256, 16), np.int32)
    for m in range(256):
        bits = [i for i in range(8) if (m >> i) & 1]
        lut[m, : len(bits)] = bits
        lut[m, 8] = len(bits)
    return jnp.asarray(lut).reshape(256 * 16)


def _widx():
    return lax.axis_index("s") * 2 + lax.axis_index("c")


def _tree_sum(v):
    pos = lax.iota(jnp.int32, 16)
    for k in (1, 2, 4, 8):
        v = v + jnp.take(v, pos ^ k)
    return v


def _bucket_edges(src, dst, lut):
    mesh = plsc.VectorSubcoreMesh(**_MESH)

    @functools.partial(
        pl.kernel,
        out_type=[
            jax.ShapeDtypeStruct((NT * CAP,), jnp.int32),
            jax.ShapeDtypeStruct((NT * 16,), jnp.int32),
        ],
        mesh=mesh,
        scratch_types=[
            [pltpu.VMEM((CHUNK,), jnp.int32)] * 2,
            [pltpu.VMEM((CHUNK,), jnp.int32)] * 2,
            pltpu.VMEM((FLUSH,), jnp.int32),
            pltpu.VMEM((256 * 16,), jnp.int32),
            pltpu.VMEM((16,), jnp.int32),
            [pltpu.SemaphoreType.DMA] * 2,
            [pltpu.SemaphoreType.DMA] * 2,
        ],
    )
    def k(src_h, dst_h, lut_h, bpack_h, counts_h, sv, dv, cp, lutv, cbuf,
          ssem, dsem):
        wid = _widx()
        lo = wid * RPB
        pos = lax.iota(jnp.int32, 16)
        pow2_8 = jnp.int32(1) << (pos & 7)
        NCH = E // CHUNK

        def s_copy(ch, p):
            return pltpu.make_async_copy(
                src_h.at[pl.ds(ch * CHUNK, CHUNK)], sv[p], ssem[p])

        def d_copy(ch, p):
            return pltpu.make_async_copy(
                dst_h.at[pl.ds(ch * CHUNK, CHUNK)], dv[p], dsem[p])

        s_copy(0, 0).start()
        d_copy(0, 0).start()
        pltpu.sync_copy(lut_h, lutv)

        def chunk_body(ch, off, p):
            @pl.when(ch + 1 < NCH)
            def _():
                s_copy(ch + 1, 1 - p).start()
                d_copy(ch + 1, 1 - p).start()

            s_copy(ch, p).wait()
            d_copy(ch, p).wait()

            def vec_body(t, cnt):
                for u in range(4):
                    sl = pl.ds(t * 64 + u * 16, 16)
                    svv = sv[p][sl]
                    dvv = dv[p][sl]
                    m = (dvv >= lo) & (dvv < lo + RPB)
                    pk = jnp.where(m, ((dvv - lo) << PKSHIFT) | svv, DUMMY_PK)
                    s = jnp.where(m, pow2_8, 0)
                    for kk in (1, 2, 4):
                        s = s + jnp.take(s, pos ^ kk)
                    mlo = s[0]
                    mhi = s[8]
                    rlo = lutv[pl.ds(mlo * 16, 16)]
                    rhi = lutv[pl.ds(mhi * 16, 16)]
                    cnt_lo = rlo[8]
                    cnt_hi = rhi[8]
                    idx_hi = jnp.clip(pos - cnt_lo, 0, 15)
                    perm = jnp.where(pos < cnt_lo, rlo,
                                     jnp.take(rhi + 8, idx_hi))
                    perm = jnp.clip(perm, 0, 15)
                    cp[pl.ds(cnt, 16)] = jnp.take(pk, perm)
                    cnt = cnt + cnt_lo + cnt_hi
                return cnt

            cnt = lax.fori_loop(0, CHUNK // 64, vec_body, 0)
            cp[pl.ds(cnt, 16)] = jnp.zeros((16,), jnp.int32) + DUMMY_PK
            cnt = (cnt + 7) & (-8)
            pltpu.sync_copy(
                cp, bpack_h.at[pl.ds(pl.multiple_of(wid * CAP + off, 8), FLUSH)])
            return off + cnt

        def chunk_pair(g, off):
            off = chunk_body(g * 2, off, 0)
            off = chunk_body(g * 2 + 1, off, 1)
            return off

        off = lax.fori_loop(0, (E // CHUNK) // 2, chunk_pair, 0)
        for q in range(4):
            cp[pl.ds(q * 16, 16)] = jnp.zeros((16,), jnp.int32) + DUMMY_PK
        pltpu.sync_copy(
            cp, bpack_h.at[pl.ds(pl.multiple_of(wid * CAP + off, 8), FLUSH)])
        cbuf[...] = jnp.zeros((16,), jnp.int32) + off
        pltpu.sync_copy(cbuf, counts_h.at[pl.ds(pl.multiple_of(wid * 16, 16), 16)])

    return k(src, dst, lut)


def _segment_max(h2, bpack, counts):
    F = h2.shape[1]
    mesh = plsc.VectorSubcoreMesh(**_MESH)

    @functools.partial(
        pl.kernel,
        out_type=jax.ShapeDtypeStruct((NPAD, F), jnp.float32),
        mesh=mesh,
        scratch_types=[
            [pltpu.VMEM((GB,), jnp.int32)] * 2,
            [pltpu.VMEM((GB,), jnp.int32)] * 2,
            [pltpu.VMEM((GB,), jnp.int32)] * 2,
            [pltpu.VMEM((GB, F), jnp.float32)] * 2,
            pltpu.VMEM((RPB + 1, F), jnp.float32),
            pltpu.VMEM((16,), jnp.int32),
            [pltpu.SemaphoreType.DMA] * 2,
            [pltpu.SemaphoreType.DMA] * 2,
        ],
    )
    def k(h2_h, bp_h, cnt_h, out_h, pkbuf, idx, lbuf, stage, acc, cbuf,
          pksem, gsem):
        wid = _widx()
        neg = jnp.full((16,), -jnp.inf, jnp.float32)

        pltpu.sync_copy(cnt_h.at[pl.ds(pl.multiple_of(wid * 16, 16), 16)], cbuf)
        m_total = cbuf[pl.ds(0, 16)][0]
        nb = (m_total + GB - 1) // GB

        def pk_copy(i, b):
            return pltpu.make_async_copy(
                bp_h.at[pl.ds(pl.multiple_of(wid * CAP + i * GB, 8), GB)],
                pkbuf[b], pksem[b])

        def gather(b):
            return pltpu.make_async_copy(h2_h.at[idx[b]], stage[b], gsem[b])

        def unpack_and_fire(b):
            for q in range(GB // 16):
                sl = pl.ds(q * 16, 16)
                pv = pkbuf[b][sl]
                idx[b][sl] = pv & PKMASK
                lbuf[b][sl] = pv >> PKSHIFT
            gather(b).start()

        @pl.when(nb > 0)
        def _():
            pk_copy(0, 0).start()

        @pl.when(nb > 1)
        def _():
            pk_copy(1, 1).start()

        @pl.when(nb > 0)
        def _():
            pk_copy(0, 0).wait()
            unpack_and_fire(0)

        def init_row(r, _):
            for kk in range(F // 16):
                acc[r, pl.ds(kk * 16, 16)] = neg
            return 0

        lax.fori_loop(0, RPB + 1, init_row, 0)

        def pair(g, _):
            for b in range(2):
                i = g * 2 + b

                @pl.when(i + 1 < nb)
                def _():
                    pk_copy(i + 1, 1 - b).wait()
                    unpack_and_fire(1 - b)

                @pl.when(i + 2 < nb)
                def _():
                    pk_copy(i + 2, b).start()

                @pl.when(i < nb)
                def _():
                    gather(b).wait()

                    def grp(q, _):
                        ldv = lbuf[b][pl.ds(q * 16, 16)]
                        for u in range(16):
                            ld = ldv[u]
                            jr = q * 16 + u
                            for kk in range(F // 16):
                                sl = pl.ds(kk * 16, 16)
                                acc[ld, sl] = jnp.maximum(acc[ld, sl],
                                                          stage[b][jr, sl])
                        return 0

                    lax.fori_loop(0, GB // 16, grp, 0)
            return 0

        lax.fori_loop(0, (nb + 1) // 2, pair, 0)

        def fix_row(r, _):
            for kk in range(F // 16):
                sl = pl.ds(kk * 16, 16)
                v = acc[r, sl]
                acc[r, sl] = jnp.where(v == -jnp.inf, 0.0, v)
            return 0

        lax.fori_loop(0, RPB, fix_row, 0)
        pltpu.sync_copy(acc.at[pl.ds(0, RPB)],
                        out_h.at[pl.ds(pl.multiple_of(wid * RPB, 8), RPB)])

    return k(h2, bpack, counts)


def _mlp_body(x_ref, w1_ref, b1_ref, g_ref, bt_ref, w2_ref, b2_ref, o_ref):
    h = jnp.dot(x_ref[...], w1_ref[...], preferred_element_type=jnp.float32)
    h = h + b1_ref[...]
    mu = jnp.mean(h, axis=1, keepdims=True)
    var = jnp.mean(jnp.square(h - mu), axis=1, keepdims=True)
    h = (h - mu) * lax.rsqrt(var + 1e-5) * g_ref[...] + bt_ref[...]
    h = jnp.maximum(h, 0.0)
    o_ref[...] = jnp.dot(h, w2_ref[...], preferred_element_type=jnp.float32) + b2_ref[...]


def _mlp(x, W1, b1, g, bt, W2, b2):
    n, c = x.shape
    BN = 400
    return pl.pallas_call(
        _mlp_body,
        grid=(n // BN,),
        in_specs=[
            pl.BlockSpec((BN, c), lambda i: (i, 0)),
            pl.BlockSpec((c, H), lambda i: (0, 0)),
            pl.BlockSpec((1, H), lambda i: (0, 0)),
            pl.BlockSpec((1, H), lambda i: (0, 0)),
            pl.BlockSpec((1, H), lambda i: (0, 0)),
            pl.BlockSpec((H, c), lambda i: (0, 0)),
            pl.BlockSpec((1, c), lambda i: (0, 0)),
        ],
        out_specs=pl.BlockSpec((BN, c), lambda i: (i, 0)),
        out_shape=jax.ShapeDtypeStruct((n, c), jnp.float32),
    )(x, W1, b1.reshape(1, H), g.reshape(1, H), bt.reshape(1, H), W2,
      b2.reshape(1, c))


def kernel(x, edge_index,
           W1_0, b1_0, g_0, bt_0, W2_0, b2_0,
           W1_1, b1_1, g_1, bt_1, W2_1, b2_1,
           W1_2, b1_2, g_2, bt_2, W2_2, b2_2):
    src = edge_index[0]
    dst = edge_index[1]
    bpack, counts = _bucket_edges(src, dst, _make_lut())
    params = [
        (W1_0, b1_0, g_0, bt_0, W2_0, b2_0),
        (W1_1, b1_1, g_1, bt_1, W2_1, b2_1),
        (W1_2, b1_2, g_2, bt_2, W2_2, b2_2),
    ]
    for W1, b1, g, bt, W2, b2 in params:
        h2 = _mlp(x, W1, b1, g, bt, W2, b2)
        c = h2.shape[1]
        if c <= 256:
            agg = _segment_max(h2, bpack, counts)[:N]
        else:
            parts = [
                _segment_max(h2[:, f0:f0 + 256], bpack, counts)[:N]
                for f0 in range(0, c, 256)
            ]
            agg = jnp.concatenate(parts, axis=1)
        x = jnp.concatenate([h2, agg], axis=1)
    return x
